# Initial kernel scaffold; baseline (speedup 1.0000x reference)
import functools

import jax
import jax.numpy as jnp
from jax import lax
from jax.experimental import pallas as pl
from jax.experimental.pallas import tpu as pltpu
from jax.experimental.pallas import tpu_sc as plsc

N_NODES = 10000
N_EDGES = 320000
NDIM_IN = 128
EDIMS = 16
NDIM_OUT = 128

NC = 2
NS = 16
NW = NC * NS
E_PER_W = N_EDGES // NW
CHUNK = 125
NCHUNK = E_PER_W // CHUNK
ROWS_PER_TILE = N_NODES // NS
EW = 32


def _sc_segment_sums(nfeats2d, src_idx, dst_idx, ef_ext, zrow_a, zrow_b):
  mesh = plsc.VectorSubcoreMesh(core_axis_name="c", subcore_axis_name="s")

  @functools.partial(
      pl.kernel,
      out_type=[
          jax.ShapeDtypeStruct((NC, N_NODES, NDIM_IN), jnp.float32),
          jax.ShapeDtypeStruct((NC, N_NODES, EW), jnp.float32),
      ],
      mesh=mesh,
      scratch_types=[
          pltpu.VMEM((N_NODES, NDIM_IN), jnp.float32, memory_space=pltpu.MemorySpace.VMEM_SHARED),
          pltpu.VMEM((N_NODES, EW), jnp.float32, memory_space=pltpu.MemorySpace.VMEM_SHARED),
          pltpu.VMEM((NCHUNK, CHUNK), jnp.int32),
          pltpu.VMEM((NCHUNK, CHUNK), jnp.int32),
          pltpu.VMEM((CHUNK, NDIM_IN), jnp.float32),
          pltpu.VMEM((CHUNK, EW), jnp.float32),
          pltpu.SemaphoreType.DMA,
      ],
  )
  def k(nf_hbm, sidx_hbm, didx_hbm, ef_hbm, za_hbm, zb_hbm,
        a_out, bd_out, a_sh, bd_sh, sidx_v, didx_v, rows_v, ef_v, sem):
    c = lax.axis_index("c")
    s = lax.axis_index("s")
    wid = c * NS + s
    pltpu.sync_copy(sidx_hbm.at[wid], sidx_v)
    pltpu.sync_copy(didx_hbm.at[wid], didx_v)
    pltpu.sync_copy(za_hbm, a_sh.at[pl.ds(s * ROWS_PER_TILE, ROWS_PER_TILE)])
    pltpu.sync_copy(zb_hbm, bd_sh.at[pl.ds(s * ROWS_PER_TILE, ROWS_PER_TILE)])
    plsc.subcore_barrier()

    def chunk_body(j, carry):
      pltpu.async_copy(nf_hbm.at[sidx_v.at[j]], rows_v, sem).wait()
      pltpu.sync_copy(ef_hbm.at[wid, j], ef_v)
      pltpu.sync_copy(rows_v, a_sh.at[didx_v.at[j]], add=True)
      pltpu.sync_copy(ef_v, bd_sh.at[didx_v.at[j]], add=True)
      return carry

    lax.fori_loop(0, NCHUNK, chunk_body, 0)
    plsc.subcore_barrier()
    pltpu.sync_copy(a_sh.at[pl.ds(s * ROWS_PER_TILE, ROWS_PER_TILE)],
                    a_out.at[c, pl.ds(s * ROWS_PER_TILE, ROWS_PER_TILE)])
    pltpu.sync_copy(bd_sh.at[pl.ds(s * ROWS_PER_TILE, ROWS_PER_TILE)],
                    bd_out.at[c, pl.ds(s * ROWS_PER_TILE, ROWS_PER_TILE)])

  return k(nfeats2d, src_idx, dst_idx, ef_ext, zrow_a, zrow_b)


def _tc_dense_body(a2, bd2, h, w1t, w2pt, wa1t, wa2t, bapp, out):
  a = a2[0] + a2[1]
  bd = bd2[0] + bd2[1]
  deg = bd[:, EDIMS:EDIMS + 1]
  msum = (jnp.dot(a, w1t[...], preferred_element_type=jnp.float32)
          + jnp.dot(bd, w2pt[...], preferred_element_type=jnp.float32))
  hn = msum / jnp.maximum(deg, 1.0)
  comb = (jnp.dot(h[...], wa1t[...], preferred_element_type=jnp.float32)
          + jnp.dot(hn, wa2t[...], preferred_element_type=jnp.float32)
          + bapp[...])
  out[...] = jnp.maximum(comb, 0.0)


def _tc_dense(a2, bd2, h, w1t, w2pt, wa1t, wa2t, bapp):
  blk = 1000
  grid = N_NODES // blk
  return pl.pallas_call(
      _tc_dense_body,
      grid=(grid,),
      in_specs=[
          pl.BlockSpec((NC, blk, NDIM_IN), lambda i: (0, i, 0)),
          pl.BlockSpec((NC, blk, EW), lambda i: (0, i, 0)),
          pl.BlockSpec((blk, NDIM_IN), lambda i: (i, 0)),
          pl.BlockSpec((NDIM_IN, NDIM_OUT), lambda i: (0, 0)),
          pl.BlockSpec((EW, NDIM_OUT), lambda i: (0, 0)),
          pl.BlockSpec((NDIM_IN, NDIM_OUT), lambda i: (0, 0)),
          pl.BlockSpec((NDIM_OUT, NDIM_OUT), lambda i: (0, 0)),
          pl.BlockSpec((1, NDIM_OUT), lambda i: (0, 0)),
      ],
      out_specs=pl.BlockSpec((blk, NDIM_OUT), lambda i: (i, 0)),
      out_shape=jax.ShapeDtypeStruct((N_NODES, NDIM_OUT), jnp.float32),
  )(a2, bd2, h, w1t, w2pt, wa1t, wa2t, bapp)


def kernel(nfeats, efeats, edge_index, W_msg_w, W_msg_b, W_apply_w, W_apply_b):
  h = nfeats[:, 0, :]
  src = edge_index[0].astype(jnp.int32).reshape(NW, NCHUNK, CHUNK)
  dst = edge_index[1].astype(jnp.int32).reshape(NW, NCHUNK, CHUNK)
  ef = efeats[:, 0, :]
  ef_ext = jnp.concatenate(
      [ef, jnp.ones((N_EDGES, 1), jnp.float32),
       jnp.zeros((N_EDGES, EW - EDIMS - 1), jnp.float32)], axis=1)
  ef_ext = ef_ext.reshape(NW, NCHUNK, CHUNK, EW)
  zrow_a = jnp.zeros((ROWS_PER_TILE, NDIM_IN), jnp.float32)
  zrow_b = jnp.zeros((ROWS_PER_TILE, EW), jnp.float32)

  a2, bd2 = _sc_segment_sums(h, src, dst, ef_ext, zrow_a, zrow_b)

  w1t = W_msg_w[:, :NDIM_IN].T
  w2pt = jnp.concatenate(
      [W_msg_w[:, NDIM_IN:].T, W_msg_b[None, :],
       jnp.zeros((EW - EDIMS - 1, NDIM_OUT), jnp.float32)], axis=0)
  wa1t = W_apply_w[:, :NDIM_IN].T
  wa2t = W_apply_w[:, NDIM_IN:].T
  out = _tc_dense(a2, bd2, h, w1t, w2pt, wa1t, wa2t, W_apply_b[None, :])
  return out[:, None, :]


# speedup vs baseline: 2.6597x; 2.6597x over previous
import functools

import jax
import jax.numpy as jnp
from jax import lax
from jax.experimental import pallas as pl
from jax.experimental.pallas import tpu as pltpu
from jax.experimental.pallas import tpu_sc as plsc

N_NODES = 10000
N_EDGES = 320000
NDIM_IN = 128
EDIMS = 16
NDIM_OUT = 128

NC = 2
NS = 16
HALF = NDIM_IN // NC
CHUNK = 128
GRP = 8
NGRP = 20
NCHUNK = GRP * NGRP
E_PAD = NS * NCHUNK * CHUNK
N_DUMMY = 10000
N_PAD = 10240
ROWS_PER_TILE = N_PAD // NS
EW = 32


def _sc_segment_sums(nf_halves, src_idx, dst_idx, ef_ext, zrow_a, zrow_b):
  mesh = plsc.VectorSubcoreMesh(core_axis_name="c", subcore_axis_name="s")

  @functools.partial(
      pl.kernel,
      out_type=[
          jax.ShapeDtypeStruct((NC, N_PAD, HALF), jnp.float32),
          jax.ShapeDtypeStruct((N_PAD, EW), jnp.float32),
      ],
      mesh=mesh,
      scratch_types=[
          pltpu.VMEM_SHARED((N_PAD, HALF), jnp.float32),
          pltpu.VMEM_SHARED((N_PAD, EW), jnp.float32),
          pltpu.VMEM((GRP, CHUNK), jnp.int32),
          pltpu.VMEM((GRP, CHUNK), jnp.int32),
          pltpu.VMEM((CHUNK, HALF), jnp.float32),
          pltpu.VMEM((CHUNK, EW), jnp.float32),
          pltpu.SemaphoreType.DMA,
      ],
      compiler_params=pltpu.CompilerParams(use_tc_tiling_on_sc=False),
  )
  def k(nf_hbm, sidx_hbm, didx_hbm, ef_hbm, za_hbm, zb_hbm,
        a_out, bd_out, a_sh, bd_sh, sidx_v, didx_v, rows_v, ef_v, sem):
    c = lax.axis_index("c")
    s = lax.axis_index("s")
    pltpu.sync_copy(za_hbm, a_sh.at[pl.ds(s * ROWS_PER_TILE, ROWS_PER_TILE)])
    pltpu.sync_copy(zb_hbm, bd_sh.at[pl.ds(s * ROWS_PER_TILE, ROWS_PER_TILE)])
    plsc.subcore_barrier()

    def group_body(g, carry):
      pltpu.sync_copy(sidx_hbm.at[s, pl.ds(g * GRP, GRP)], sidx_v)
      pltpu.sync_copy(didx_hbm.at[s, pl.ds(g * GRP, GRP)], didx_v)

      def chunk_body(j, carry2):
        pltpu.async_copy(nf_hbm.at[c].at[sidx_v.at[j]], rows_v, sem).wait()
        pltpu.sync_copy(rows_v, a_sh.at[didx_v.at[j]], add=True)

        @pl.when(c == 0)
        def _():
          pltpu.sync_copy(ef_hbm.at[s, g * GRP + j], ef_v)
          pltpu.sync_copy(ef_v, bd_sh.at[didx_v.at[j]], add=True)

        return carry2

      lax.fori_loop(0, GRP, chunk_body, carry)
      return carry

    lax.fori_loop(0, NGRP, group_body, 0)
    plsc.subcore_barrier()
    pltpu.sync_copy(a_sh.at[pl.ds(s * ROWS_PER_TILE, ROWS_PER_TILE)],
                    a_out.at[c, pl.ds(s * ROWS_PER_TILE, ROWS_PER_TILE)])

    @pl.when(c == 0)
    def _():
      pltpu.sync_copy(bd_sh.at[pl.ds(s * ROWS_PER_TILE, ROWS_PER_TILE)],
                      bd_out.at[pl.ds(s * ROWS_PER_TILE, ROWS_PER_TILE)])

  return k(nf_halves, src_idx, dst_idx, ef_ext, zrow_a, zrow_b)


def _tc_dense_body(a2, bd, h, w1t, w2pt, wa1t, wa2t, bapp, out):
  a = jnp.concatenate([a2[0], a2[1]], axis=-1)
  bdv = bd[...]
  deg = bdv[:, EDIMS:EDIMS + 1]
  msum = (jnp.dot(a, w1t[...], preferred_element_type=jnp.float32)
          + jnp.dot(bdv, w2pt[...], preferred_element_type=jnp.float32))
  hn = msum / jnp.maximum(deg, 1.0)
  comb = (jnp.dot(h[...], wa1t[...], preferred_element_type=jnp.float32)
          + jnp.dot(hn, wa2t[...], preferred_element_type=jnp.float32)
          + bapp[...])
  out[...] = jnp.maximum(comb, 0.0)


def _tc_dense(a2, bd, h, w1t, w2pt, wa1t, wa2t, bapp):
  blk = 1000
  grid = N_NODES // blk
  return pl.pallas_call(
      _tc_dense_body,
      grid=(grid,),
      in_specs=[
          pl.BlockSpec((NC, blk, HALF), lambda i: (0, i, 0)),
          pl.BlockSpec((blk, EW), lambda i: (i, 0)),
          pl.BlockSpec((blk, NDIM_IN), lambda i: (i, 0)),
          pl.BlockSpec((NDIM_IN, NDIM_OUT), lambda i: (0, 0)),
          pl.BlockSpec((EW, NDIM_OUT), lambda i: (0, 0)),
          pl.BlockSpec((NDIM_IN, NDIM_OUT), lambda i: (0, 0)),
          pl.BlockSpec((NDIM_OUT, NDIM_OUT), lambda i: (0, 0)),
          pl.BlockSpec((1, NDIM_OUT), lambda i: (0, 0)),
      ],
      out_specs=pl.BlockSpec((blk, NDIM_OUT), lambda i: (i, 0)),
      out_shape=jax.ShapeDtypeStruct((N_NODES, NDIM_OUT), jnp.float32),
  )(a2, bd, h, w1t, w2pt, wa1t, wa2t, bapp)


def kernel(nfeats, efeats, edge_index, W_msg_w, W_msg_b, W_apply_w, W_apply_b):
  h = nfeats[:, 0, :]
  nf_halves = h.reshape(N_NODES, NC, HALF).transpose(1, 0, 2)
  pad = E_PAD - N_EDGES
  src = jnp.concatenate(
      [edge_index[0].astype(jnp.int32), jnp.zeros((pad,), jnp.int32)])
  src = src.reshape(NS, NCHUNK, CHUNK)
  dst = jnp.concatenate(
      [edge_index[1].astype(jnp.int32),
       jnp.full((pad,), N_DUMMY, jnp.int32)])
  dst = dst.reshape(NS, NCHUNK, CHUNK)
  ef = efeats[:, 0, :]
  ef_ext = jnp.concatenate(
      [ef, jnp.ones((N_EDGES, 1), jnp.float32),
       jnp.zeros((N_EDGES, EW - EDIMS - 1), jnp.float32)], axis=1)
  ef_ext = jnp.concatenate([ef_ext, jnp.zeros((pad, EW), jnp.float32)], axis=0)
  ef_ext = ef_ext.reshape(NS, NCHUNK, CHUNK, EW)
  zrow_a = jnp.zeros((ROWS_PER_TILE, HALF), jnp.float32)
  zrow_b = jnp.zeros((ROWS_PER_TILE, EW), jnp.float32)

  a2, bd = _sc_segment_sums(nf_halves, src, dst, ef_ext, zrow_a, zrow_b)

  w1t = W_msg_w[:, :NDIM_IN].T
  w2pt = jnp.concatenate(
      [W_msg_w[:, NDIM_IN:].T, W_msg_b[None, :],
       jnp.zeros((EW - EDIMS - 1, NDIM_OUT), jnp.float32)], axis=0)
  wa1t = W_apply_w[:, :NDIM_IN].T
  wa2t = W_apply_w[:, NDIM_IN:].T
  out = _tc_dense(a2, bd, h, w1t, w2pt, wa1t, wa2t, W_apply_b[None, :])
  return out[:, None, :]


# speedup vs baseline: 3.7969x; 1.4276x over previous
import functools

import jax
import jax.numpy as jnp
from jax import lax
from jax.experimental import pallas as pl
from jax.experimental.pallas import tpu as pltpu
from jax.experimental.pallas import tpu_sc as plsc

N_NODES = 10000
N_EDGES = 320000
NDIM_IN = 128
EDIMS = 16
NDIM_OUT = 128

NC = 2
NS = 16
HALF = NDIM_IN // NC
CHUNK = 128
GRP = 8
NGRP = 20
NCHUNK = GRP * NGRP
NCH2 = NCHUNK // NC
E_PAD = NS * NCHUNK * CHUNK
N_DUMMY = 10000
N_PAD = 10240
ROWS_PER_TILE = N_PAD // NS
NBUF = 4
PD = 2


def _sc_segment_sums(nf_halves, src_idx, dst_idx, ef, ones_blk,
                     zrow_a, zrow_e):
  mesh = plsc.VectorSubcoreMesh(core_axis_name="c", subcore_axis_name="s")

  @functools.partial(
      pl.kernel,
      out_type=[
          jax.ShapeDtypeStruct((NC, N_PAD, HALF), jnp.float32),
          jax.ShapeDtypeStruct((NC, N_PAD, EDIMS), jnp.float32),
          jax.ShapeDtypeStruct((NC, N_PAD, EDIMS), jnp.float32),
      ],
      mesh=mesh,
      scratch_types=[
          pltpu.VMEM_SHARED((N_PAD, HALF), jnp.float32),
          pltpu.VMEM_SHARED((N_PAD, EDIMS), jnp.float32),
          pltpu.VMEM_SHARED((N_PAD, EDIMS), jnp.float32),
          pltpu.VMEM((4, GRP, CHUNK), jnp.int32),
          pltpu.VMEM((4, GRP, CHUNK), jnp.int32),
          pltpu.VMEM((NBUF, CHUNK, HALF), jnp.float32),
          pltpu.VMEM((2, CHUNK, EDIMS), jnp.float32),
          pltpu.VMEM((CHUNK, EDIMS), jnp.float32),
          pltpu.SemaphoreType.DMA,
          pltpu.SemaphoreType.DMA((NBUF,)),
          pltpu.SemaphoreType.DMA((NBUF,)),
          pltpu.SemaphoreType.DMA((2,)),
          pltpu.SemaphoreType.DMA((2,)),
          pltpu.SemaphoreType.DMA,
      ],
      compiler_params=pltpu.CompilerParams(use_tc_tiling_on_sc=False),
  )
  def k(nf_hbm, sidx_hbm, didx_hbm, ef_hbm, ones_hbm, za_hbm, ze_hbm,
        a_out, bef_out, deg_out,
        a_sh, bef_sh, deg_sh, sidx_v, didx_v, rows_v, ef_v, ones_v,
        sem_i, sem_g, sem_s, sem_e, sem_f, sem_o):
    c = lax.axis_index("c")
    s = lax.axis_index("s")
    row0 = s * ROWS_PER_TILE
    pltpu.sync_copy(za_hbm, a_sh.at[pl.ds(row0, ROWS_PER_TILE)])
    pltpu.sync_copy(ze_hbm, bef_sh.at[pl.ds(row0, ROWS_PER_TILE)])
    pltpu.sync_copy(ze_hbm, deg_sh.at[pl.ds(row0, ROWS_PER_TILE)])
    pltpu.sync_copy(ones_hbm, ones_v)
    plsc.subcore_barrier()

    pltpu.sync_copy(sidx_hbm.at[s, pl.ds(0, GRP)], sidx_v.at[0])
    pltpu.sync_copy(didx_hbm.at[s, pl.ds(0, GRP)], didx_v.at[0])
    pltpu.async_copy(sidx_hbm.at[s, pl.ds(GRP, GRP)], sidx_v.at[1], sem_i)
    pltpu.async_copy(didx_hbm.at[s, pl.ds(GRP, GRP)], didx_v.at[1], sem_i)

    def p1_round(r, carry):
      for b in range(NBUF):
        if b == 0:

          @pl.when((lax.rem(r, 2) == 0) & (r > 0))
          def _():
            g = r // 2
            slot = lax.rem(g, 4)
            pltpu.make_async_copy(
                sidx_hbm.at[s, pl.ds(0, GRP)], sidx_v.at[slot], sem_i).wait()
            pltpu.make_async_copy(
                didx_hbm.at[s, pl.ds(0, GRP)], didx_v.at[slot], sem_i).wait()

            @pl.when(g + 1 < NGRP)
            def _():
              nslot = lax.rem(g + 1, 4)
              pltpu.async_copy(
                  sidx_hbm.at[s, pl.ds((g + 1) * GRP, GRP)], sidx_v.at[nslot],
                  sem_i)
              pltpu.async_copy(
                  didx_hbm.at[s, pl.ds((g + 1) * GRP, GRP)], didx_v.at[nslot],
                  sem_i)

        @pl.when(r >= 1)
        def _():
          pltpu.make_async_copy(
              rows_v.at[b], a_sh.at[pl.ds(0, CHUNK)], sem_s.at[b]).wait()

        idx = sidx_v.at[lax.rem(r // 2, 4), 4 * lax.rem(r, 2) + b]
        pltpu.async_copy(nf_hbm.at[c].at[idx], rows_v.at[b], sem_g.at[b])

        bs = (b + PD) % NBUF
        jj = r * NBUF + b - PD

        def scat(jj=jj, bs=bs):
          pltpu.make_async_copy(
              za_hbm.at[pl.ds(0, CHUNK)], rows_v.at[bs], sem_g.at[bs]).wait()
          idx2 = didx_v.at[lax.rem(jj // GRP, 4), lax.rem(jj, GRP)]
          pltpu.async_copy(rows_v.at[bs], a_sh.at[idx2], sem_s.at[bs],
                           add=True)

        if b < PD:
          pl.when(r >= 1)(scat)
        else:
          scat()
      return carry

    lax.fori_loop(0, NCHUNK // NBUF, p1_round, 0)
    for b in range(PD):
      bs = (b + PD) % NBUF
      jj = NCHUNK + b - PD
      pltpu.make_async_copy(
          za_hbm.at[pl.ds(0, CHUNK)], rows_v.at[bs], sem_g.at[bs]).wait()
      idx2 = didx_v.at[(jj // GRP) % 4, jj % GRP]
      pltpu.async_copy(rows_v.at[bs], a_sh.at[idx2], sem_s.at[bs], add=True)
    for b in range(NBUF):
      pltpu.make_async_copy(
          rows_v.at[b], a_sh.at[pl.ds(0, CHUNK)], sem_s.at[b]).wait()

    ch0 = c * NCH2
    pltpu.sync_copy(didx_hbm.at[s, pl.ds(ch0, GRP)], didx_v.at[0])
    pltpu.async_copy(didx_hbm.at[s, pl.ds(ch0 + GRP, GRP)], didx_v.at[1],
                     sem_i)

    def p2_round(r, carry):
      for b in range(2):
        if b == 0:

          @pl.when((lax.rem(r, 4) == 0) & (r > 0))
          def _():
            lg = r // 4
            slot = lax.rem(lg, 4)
            pltpu.make_async_copy(
                didx_hbm.at[s, pl.ds(0, GRP)], didx_v.at[slot], sem_i).wait()

            @pl.when(lg + 1 < NCH2 // GRP)
            def _():
              nslot = lax.rem(lg + 1, 4)
              pltpu.async_copy(
                  didx_hbm.at[s, pl.ds(ch0 + (lg + 1) * GRP, GRP)],
                  didx_v.at[nslot], sem_i)

        @pl.when(r >= 1)
        def _():
          pltpu.make_async_copy(
              ef_v.at[b], bef_sh.at[pl.ds(0, CHUNK)], sem_f.at[b]).wait()

        k = 2 * r + b
        pltpu.async_copy(ef_hbm.at[s, ch0 + k], ef_v.at[b], sem_e.at[b])

        b2 = 1 - b
        k2 = 2 * r + b - 1

        def scat2(k2=k2, b2=b2, ones_lag=(r >= 2 if b == 0 else r >= 1)):
          pltpu.make_async_copy(
              ef_hbm.at[s, 0], ef_v.at[b2], sem_e.at[b2]).wait()
          idx3 = didx_v.at[lax.rem(k2 // GRP, 4), lax.rem(k2, GRP)]
          pltpu.async_copy(ef_v.at[b2], bef_sh.at[idx3], sem_f.at[b2],
                           add=True)

          def ones_wait():
            pltpu.make_async_copy(
                ones_hbm, deg_sh.at[pl.ds(0, CHUNK)], sem_o).wait()

          pl.when(ones_lag)(ones_wait)
          pltpu.async_copy(ones_v, deg_sh.at[idx3], sem_o, add=True)

        if b == 0:
          pl.when(r >= 1)(scat2)
        else:
          scat2()
      return carry

    lax.fori_loop(0, NCH2 // 2, p2_round, 0)
    pltpu.make_async_copy(ef_hbm.at[s, 0], ef_v.at[1], sem_e.at[1]).wait()
    idx4 = didx_v.at[((NCH2 - 1) // GRP) % 4, (NCH2 - 1) % GRP]
    pltpu.async_copy(ef_v.at[1], bef_sh.at[idx4], sem_f.at[1], add=True)
    pltpu.make_async_copy(ones_hbm, deg_sh.at[pl.ds(0, CHUNK)], sem_o).wait()
    pltpu.async_copy(ones_v, deg_sh.at[idx4], sem_o, add=True)
    for b in range(2):
      pltpu.make_async_copy(
          ef_v.at[b], bef_sh.at[pl.ds(0, CHUNK)], sem_f.at[b]).wait()
      pltpu.make_async_copy(
          ones_hbm, deg_sh.at[pl.ds(0, CHUNK)], sem_o).wait()

    plsc.subcore_barrier()
    pltpu.sync_copy(a_sh.at[pl.ds(row0, ROWS_PER_TILE)],
                    a_out.at[c, pl.ds(row0, ROWS_PER_TILE)])
    pltpu.sync_copy(bef_sh.at[pl.ds(row0, ROWS_PER_TILE)],
                    bef_out.at[c, pl.ds(row0, ROWS_PER_TILE)])
    pltpu.sync_copy(deg_sh.at[pl.ds(row0, ROWS_PER_TILE)],
                    deg_out.at[c, pl.ds(row0, ROWS_PER_TILE)])

  return k(nf_halves, src_idx, dst_idx, ef, ones_blk, zrow_a, zrow_e)


def _tc_dense_body(a2, bef2, deg2, h, w1t, w2t, wa1t, wa2t, bm, bapp, out):
  a = jnp.concatenate([a2[0], a2[1]], axis=-1)
  bef = bef2[0] + bef2[1]
  deg = (deg2[0] + deg2[1])[:, :1]
  msum = (jnp.dot(a, w1t[...], preferred_element_type=jnp.float32)
          + jnp.dot(bef, w2t[...], preferred_element_type=jnp.float32)
          + deg * bm[...])
  hn = msum / jnp.maximum(deg, 1.0)
  comb = (jnp.dot(h[...], wa1t[...], preferred_element_type=jnp.float32)
          + jnp.dot(hn, wa2t[...], preferred_element_type=jnp.float32)
          + bapp[...])
  out[...] = jnp.maximum(comb, 0.0)


def _tc_dense(a2, bef2, deg2, h, w1t, w2t, wa1t, wa2t, bm, bapp):
  blk = 1000
  grid = N_NODES // blk
  return pl.pallas_call(
      _tc_dense_body,
      grid=(grid,),
      in_specs=[
          pl.BlockSpec((NC, blk, HALF), lambda i: (0, i, 0)),
          pl.BlockSpec((NC, blk, EDIMS), lambda i: (0, i, 0)),
          pl.BlockSpec((NC, blk, EDIMS), lambda i: (0, i, 0)),
          pl.BlockSpec((blk, NDIM_IN), lambda i: (i, 0)),
          pl.BlockSpec((NDIM_IN, NDIM_OUT), lambda i: (0, 0)),
          pl.BlockSpec((EDIMS, NDIM_OUT), lambda i: (0, 0)),
          pl.BlockSpec((NDIM_IN, NDIM_OUT), lambda i: (0, 0)),
          pl.BlockSpec((NDIM_OUT, NDIM_OUT), lambda i: (0, 0)),
          pl.BlockSpec((1, NDIM_OUT), lambda i: (0, 0)),
          pl.BlockSpec((1, NDIM_OUT), lambda i: (0, 0)),
      ],
      out_specs=pl.BlockSpec((blk, NDIM_OUT), lambda i: (i, 0)),
      out_shape=jax.ShapeDtypeStruct((N_NODES, NDIM_OUT), jnp.float32),
  )(a2, bef2, deg2, h, w1t, w2t, wa1t, wa2t, bm, bapp)


def kernel(nfeats, efeats, edge_index, W_msg_w, W_msg_b, W_apply_w, W_apply_b):
  h = nfeats[:, 0, :]
  nf_halves = h.reshape(N_NODES, NC, HALF).transpose(1, 0, 2)
  pad = E_PAD - N_EDGES
  src = jnp.concatenate(
      [edge_index[0].astype(jnp.int32), jnp.zeros((pad,), jnp.int32)])
  src = src.reshape(NS, NCHUNK, CHUNK)
  dst = jnp.concatenate(
      [edge_index[1].astype(jnp.int32),
       jnp.full((pad,), N_DUMMY, jnp.int32)])
  dst = dst.reshape(NS, NCHUNK, CHUNK)
  ef = jnp.concatenate(
      [efeats[:, 0, :], jnp.zeros((pad, EDIMS), jnp.float32)], axis=0)
  ef = ef.reshape(NS, NCHUNK, CHUNK, EDIMS)
  ones_blk = jnp.ones((CHUNK, EDIMS), jnp.float32)
  zrow_a = jnp.zeros((ROWS_PER_TILE, HALF), jnp.float32)
  zrow_e = jnp.zeros((ROWS_PER_TILE, EDIMS), jnp.float32)

  a2, bef2, deg2 = _sc_segment_sums(nf_halves, src, dst, ef, ones_blk,
                                    zrow_a, zrow_e)

  w1t = W_msg_w[:, :NDIM_IN].T
  w2t = W_msg_w[:, NDIM_IN:].T
  wa1t = W_apply_w[:, :NDIM_IN].T
  wa2t = W_apply_w[:, NDIM_IN:].T
  out = _tc_dense(a2, bef2, deg2, h, w1t, w2t, wa1t, wa2t,
                  W_msg_b[None, :], W_apply_b[None, :])
  return out[:, None, :]


# speedup vs baseline: 3.9700x; 1.0456x over previous
import functools

import jax
import jax.numpy as jnp
from jax import lax
from jax.experimental import pallas as pl
from jax.experimental.pallas import tpu as pltpu
from jax.experimental.pallas import tpu_sc as plsc

N_NODES = 10000
N_EDGES = 320000
NDIM_IN = 128
EDIMS = 16
NDIM_OUT = 128

NC = 2
NS = 16
HALF = NDIM_IN // NC
CHUNK = 128
GRP = 8
NGRP = 20
NCHUNK = GRP * NGRP
NCH2 = NCHUNK // NC
E_PAD = NS * NCHUNK * CHUNK
N_DUMMY = 10000
N_PAD = 10240
ROWS_PER_TILE = N_PAD // NS
NBUF = 4
PD = 2


def _sc_segment_sums(nf_halves, src_idx, dst_idx, ef, ones_blk,
                     zrow_a, zrow_e):
  mesh = plsc.VectorSubcoreMesh(core_axis_name="c", subcore_axis_name="s")

  @functools.partial(
      pl.kernel,
      out_type=[
          jax.ShapeDtypeStruct((NC, N_PAD, HALF), jnp.float32),
          jax.ShapeDtypeStruct((NC, N_PAD, EDIMS), jnp.float32),
          jax.ShapeDtypeStruct((NC, N_PAD, EDIMS), jnp.float32),
      ],
      mesh=mesh,
      scratch_types=[
          pltpu.VMEM_SHARED((N_PAD, HALF), jnp.float32),
          pltpu.VMEM_SHARED((N_PAD, EDIMS), jnp.float32),
          pltpu.VMEM_SHARED((N_PAD, EDIMS), jnp.float32),
          pltpu.VMEM((4, GRP, CHUNK), jnp.int32),
          pltpu.VMEM((4, GRP, CHUNK), jnp.int32),
          pltpu.VMEM((NBUF, CHUNK, HALF), jnp.float32),
          pltpu.VMEM((2, CHUNK, EDIMS), jnp.float32),
          pltpu.VMEM((CHUNK, EDIMS), jnp.float32),
          pltpu.SemaphoreType.DMA,
          pltpu.SemaphoreType.DMA((NBUF,)),
          pltpu.SemaphoreType.DMA((NBUF,)),
          pltpu.SemaphoreType.DMA((2,)),
          pltpu.SemaphoreType.DMA((2,)),
          pltpu.SemaphoreType.DMA,
      ],
      compiler_params=pltpu.CompilerParams(use_tc_tiling_on_sc=False),
  )
  def k(nf_hbm, sidx_hbm, didx_hbm, ef_hbm, ones_hbm, za_hbm, ze_hbm,
        a_out, bef_out, deg_out,
        a_sh, bef_sh, deg_sh, sidx_v, didx_v, rows_v, ef_v, ones_v,
        sem_i, sem_g, sem_s, sem_e, sem_f, sem_o):
    c = lax.axis_index("c")
    s = lax.axis_index("s")
    row0 = s * ROWS_PER_TILE
    pltpu.sync_copy(za_hbm, a_sh.at[pl.ds(row0, ROWS_PER_TILE)])
    pltpu.sync_copy(ze_hbm, bef_sh.at[pl.ds(row0, ROWS_PER_TILE)])
    pltpu.sync_copy(ze_hbm, deg_sh.at[pl.ds(row0, ROWS_PER_TILE)])
    pltpu.sync_copy(ones_hbm, ones_v)
    plsc.subcore_barrier()

    pltpu.sync_copy(sidx_hbm.at[s, pl.ds(0, GRP)], sidx_v.at[0])
    pltpu.sync_copy(didx_hbm.at[s, pl.ds(0, GRP)], didx_v.at[0])
    pltpu.async_copy(sidx_hbm.at[s, pl.ds(GRP, GRP)], sidx_v.at[1], sem_i)
    pltpu.async_copy(didx_hbm.at[s, pl.ds(GRP, GRP)], didx_v.at[1], sem_i)

    def p1_round(r, carry):
      for b in range(NBUF):
        if b == 0:

          @pl.when((lax.rem(r, 2) == 0) & (r > 0))
          def _():
            g = r // 2
            slot = lax.rem(g, 4)
            pltpu.make_async_copy(
                sidx_hbm.at[s, pl.ds(0, GRP)], sidx_v.at[slot], sem_i).wait()
            pltpu.make_async_copy(
                didx_hbm.at[s, pl.ds(0, GRP)], didx_v.at[slot], sem_i).wait()

            @pl.when(g + 1 < NGRP)
            def _():
              nslot = lax.rem(g + 1, 4)
              pltpu.async_copy(
                  sidx_hbm.at[s, pl.ds((g + 1) * GRP, GRP)], sidx_v.at[nslot],
                  sem_i)
              pltpu.async_copy(
                  didx_hbm.at[s, pl.ds((g + 1) * GRP, GRP)], didx_v.at[nslot],
                  sem_i)

        @pl.when(r >= 1)
        def _():
          pltpu.make_async_copy(
              rows_v.at[b], a_sh.at[pl.ds(0, CHUNK)], sem_s.at[b]).wait()

        idx = sidx_v.at[lax.rem(r // 2, 4), 4 * lax.rem(r, 2) + b]
        pltpu.async_copy(nf_hbm.at[c].at[idx], rows_v.at[b], sem_g.at[b])

        bs = (b + PD) % NBUF
        jj = r * NBUF + b - PD

        def scat(jj=jj, bs=bs):
          pltpu.make_async_copy(
              za_hbm.at[pl.ds(0, CHUNK)], rows_v.at[bs], sem_g.at[bs]).wait()
          idx2 = didx_v.at[lax.rem(jj // GRP, 4), lax.rem(jj, GRP)]
          pltpu.async_copy(rows_v.at[bs], a_sh.at[idx2], sem_s.at[bs],
                           add=True)

        if b < PD:
          pl.when(r >= 1)(scat)
        else:
          scat()

        efb = (b // 2) % 2

        @pl.when(c == (b % 2))
        def _():

          @pl.when(r >= 1)
          def _():
            pltpu.make_async_copy(
                ef_v.at[efb], bef_sh.at[pl.ds(0, CHUNK)], sem_f.at[efb]).wait()

          pltpu.async_copy(ef_hbm.at[s, r * NBUF + b], ef_v.at[efb],
                           sem_e.at[efb])

          def efscat(je=r * NBUF + b - 2, eb=1 - efb,
                     ones_lag=(r >= 2 if b < 2 else r >= 1)):
            pltpu.make_async_copy(
                ef_hbm.at[s, 0], ef_v.at[eb], sem_e.at[eb]).wait()
            idx3 = didx_v.at[lax.rem(je // GRP, 4), lax.rem(je, GRP)]
            pltpu.async_copy(ef_v.at[eb], bef_sh.at[idx3], sem_f.at[eb],
                             add=True)

            def ones_wait():
              pltpu.make_async_copy(
                  ones_hbm, deg_sh.at[pl.ds(0, CHUNK)], sem_o).wait()

            pl.when(ones_lag)(ones_wait)
            pltpu.async_copy(ones_v, deg_sh.at[idx3], sem_o, add=True)

          if b < 2:
            pl.when(r >= 1)(efscat)
          else:
            efscat()
      return carry

    lax.fori_loop(0, NCHUNK // NBUF, p1_round, 0)
    for b in range(PD):
      bs = (b + PD) % NBUF
      jj = NCHUNK + b - PD
      pltpu.make_async_copy(
          za_hbm.at[pl.ds(0, CHUNK)], rows_v.at[bs], sem_g.at[bs]).wait()
      idx2 = didx_v.at[(jj // GRP) % 4, jj % GRP]
      pltpu.async_copy(rows_v.at[bs], a_sh.at[idx2], sem_s.at[bs], add=True)
    for b in range(NBUF):
      pltpu.make_async_copy(
          rows_v.at[b], a_sh.at[pl.ds(0, CHUNK)], sem_s.at[b]).wait()

    je = NCHUNK - 2 + c
    pltpu.make_async_copy(ef_hbm.at[s, 0], ef_v.at[1], sem_e.at[1]).wait()
    idx5 = didx_v.at[lax.rem(je // GRP, 4), lax.rem(je, GRP)]
    pltpu.async_copy(ef_v.at[1], bef_sh.at[idx5], sem_f.at[1], add=True)
    pltpu.make_async_copy(ones_hbm, deg_sh.at[pl.ds(0, CHUNK)], sem_o).wait()
    pltpu.async_copy(ones_v, deg_sh.at[idx5], sem_o, add=True)
    for eb in range(2):
      pltpu.make_async_copy(
          ef_v.at[eb], bef_sh.at[pl.ds(0, CHUNK)], sem_f.at[eb]).wait()
      pltpu.make_async_copy(
          ones_hbm, deg_sh.at[pl.ds(0, CHUNK)], sem_o).wait()

    plsc.subcore_barrier()
    pltpu.sync_copy(a_sh.at[pl.ds(row0, ROWS_PER_TILE)],
                    a_out.at[c, pl.ds(row0, ROWS_PER_TILE)])
    pltpu.sync_copy(bef_sh.at[pl.ds(row0, ROWS_PER_TILE)],
                    bef_out.at[c, pl.ds(row0, ROWS_PER_TILE)])
    pltpu.sync_copy(deg_sh.at[pl.ds(row0, ROWS_PER_TILE)],
                    deg_out.at[c, pl.ds(row0, ROWS_PER_TILE)])

  return k(nf_halves, src_idx, dst_idx, ef, ones_blk, zrow_a, zrow_e)


def _tc_dense_body(a2, bef2, deg2, h, w1t, w2t, wa1t, wa2t, bm, bapp, out):
  a = jnp.concatenate([a2[0], a2[1]], axis=-1)
  bef = bef2[0] + bef2[1]
  deg = (deg2[0] + deg2[1])[:, :1]
  msum = (jnp.dot(a, w1t[...], preferred_element_type=jnp.float32)
          + jnp.dot(bef, w2t[...], preferred_element_type=jnp.float32)
          + deg * bm[...])
  hn = msum / jnp.maximum(deg, 1.0)
  comb = (jnp.dot(h[...], wa1t[...], preferred_element_type=jnp.float32)
          + jnp.dot(hn, wa2t[...], preferred_element_type=jnp.float32)
          + bapp[...])
  out[...] = jnp.maximum(comb, 0.0)


def _tc_dense(a2, bef2, deg2, h, w1t, w2t, wa1t, wa2t, bm, bapp):
  blk = 1000
  grid = N_NODES // blk
  return pl.pallas_call(
      _tc_dense_body,
      grid=(grid,),
      in_specs=[
          pl.BlockSpec((NC, blk, HALF), lambda i: (0, i, 0)),
          pl.BlockSpec((NC, blk, EDIMS), lambda i: (0, i, 0)),
          pl.BlockSpec((NC, blk, EDIMS), lambda i: (0, i, 0)),
          pl.BlockSpec((blk, NDIM_IN), lambda i: (i, 0)),
          pl.BlockSpec((NDIM_IN, NDIM_OUT), lambda i: (0, 0)),
          pl.BlockSpec((EDIMS, NDIM_OUT), lambda i: (0, 0)),
          pl.BlockSpec((NDIM_IN, NDIM_OUT), lambda i: (0, 0)),
          pl.BlockSpec((NDIM_OUT, NDIM_OUT), lambda i: (0, 0)),
          pl.BlockSpec((1, NDIM_OUT), lambda i: (0, 0)),
          pl.BlockSpec((1, NDIM_OUT), lambda i: (0, 0)),
      ],
      out_specs=pl.BlockSpec((blk, NDIM_OUT), lambda i: (i, 0)),
      out_shape=jax.ShapeDtypeStruct((N_NODES, NDIM_OUT), jnp.float32),
  )(a2, bef2, deg2, h, w1t, w2t, wa1t, wa2t, bm, bapp)


def kernel(nfeats, efeats, edge_index, W_msg_w, W_msg_b, W_apply_w, W_apply_b):
  h = nfeats[:, 0, :]
  nf_halves = h.reshape(N_NODES, NC, HALF).transpose(1, 0, 2)
  pad = E_PAD - N_EDGES
  src = jnp.concatenate(
      [edge_index[0].astype(jnp.int32), jnp.zeros((pad,), jnp.int32)])
  src = src.reshape(NS, NCHUNK, CHUNK)
  dst = jnp.concatenate(
      [edge_index[1].astype(jnp.int32),
       jnp.full((pad,), N_DUMMY, jnp.int32)])
  dst = dst.reshape(NS, NCHUNK, CHUNK)
  ef = jnp.concatenate(
      [efeats[:, 0, :], jnp.zeros((pad, EDIMS), jnp.float32)], axis=0)
  ef = ef.reshape(NS, NCHUNK, CHUNK, EDIMS)
  ones_blk = jnp.ones((CHUNK, EDIMS), jnp.float32)
  zrow_a = jnp.zeros((ROWS_PER_TILE, HALF), jnp.float32)
  zrow_e = jnp.zeros((ROWS_PER_TILE, EDIMS), jnp.float32)

  a2, bef2, deg2 = _sc_segment_sums(nf_halves, src, dst, ef, ones_blk,
                                    zrow_a, zrow_e)

  w1t = W_msg_w[:, :NDIM_IN].T
  w2t = W_msg_w[:, NDIM_IN:].T
  wa1t = W_apply_w[:, :NDIM_IN].T
  wa2t = W_apply_w[:, NDIM_IN:].T
  out = _tc_dense(a2, bef2, deg2, h, w1t, w2t, wa1t, wa2t,
                  W_msg_b[None, :], W_apply_b[None, :])
  return out[:, None, :]
